# baseline (speedup 1.0000x reference)
import functools

import jax
import jax.numpy as jnp
from jax import lax
from jax.experimental import pallas as pl
from jax.experimental.pallas import tpu as pltpu
from jax.experimental.pallas import tpu_sc as plsc

N = 16384
B = 256
LANES = 16
CHUNK = 96
NCHUNKS = 88

_SPLAT_DNUMS = lax.GatherDimensionNumbers(
    offset_dims=(), collapsed_slice_dims=(0,), start_index_map=(0,))


def _diff_t_body(a_ref, b_ref, o_ref):
    o_ref[...] = (a_ref[...] - b_ref[...]).T


def _diff_t(v1, v2):
    blk = 4096
    return pl.pallas_call(
        _diff_t_body,
        grid=(N // blk,),
        in_specs=[
            pl.BlockSpec((B, blk), lambda i: (0, i)),
            pl.BlockSpec((B, blk), lambda i: (0, i)),
        ],
        out_specs=pl.BlockSpec((blk, B), lambda i: (i, 0)),
        out_shape=jax.ShapeDtypeStruct((N, B), jnp.float32),
    )(v1, v2)


def _make_sc_loss(nw, nc):
    mesh = plsc.VectorSubcoreMesh(core_axis_name="c", subcore_axis_name="s")
    n_slices = B // LANES
    per_w = NCHUNKS * CHUNK

    @functools.partial(
        pl.kernel,
        mesh=mesh,
        out_type=jax.ShapeDtypeStruct((nw, LANES), jnp.float32),
        scratch_types=[
            pltpu.VMEM((per_w,), jnp.int32),
            pltpu.VMEM((per_w,), jnp.int32),
            pltpu.VMEM((CHUNK,), jnp.float32),
            pltpu.VMEM((CHUNK,), jnp.float32),
            pltpu.VMEM((CHUNK * LANES,), jnp.float32),
            pltpu.VMEM((CHUNK * LANES,), jnp.float32),
            pltpu.VMEM((CHUNK, B), jnp.float32),
            pltpu.VMEM((CHUNK, B), jnp.float32),
            pltpu.VMEM((CHUNK, B), jnp.float32),
            pltpu.VMEM((CHUNK, B), jnp.float32),
            pltpu.VMEM((LANES,), jnp.float32),
            pltpu.SemaphoreType.DMA,
            pltpu.SemaphoreType.DMA,
            pltpu.SemaphoreType.DMA,
            pltpu.SemaphoreType.DMA,
            pltpu.SemaphoreType.DMA,
            pltpu.SemaphoreType.DMA,
        ],
    )
    def sc_loss(table_hbm, rows_hbm, cols_hbm, vals_hbm, out_hbm,
                ridx_all, cidx_all, vraw0, vraw1, vexp0, vexp1,
                r0, c0, r1, c1, out_v,
                semv0, semr0, semc0, semv1, semr1, semc1):
        cid = lax.axis_index("c")
        sid = lax.axis_index("s")
        wid = sid * nc + cid
        elem_base = wid * per_w
        chunk_base = wid * NCHUNKS

        pltpu.sync_copy(rows_hbm.at[pl.ds(elem_base, per_w)], ridx_all)
        pltpu.sync_copy(cols_hbm.at[pl.ds(elem_base, per_w)], cidx_all)

        bufs = [(vraw0, vexp0, r0, c0, semv0, semr0, semc0),
                (vraw1, vexp1, r1, c1, semv1, semr1, semc1)]

        def issue(ci, b):
            vb, _, rb, cb, semv, semr, semc = bufs[b]
            pltpu.async_copy(vals_hbm.at[chunk_base + ci], vb, semv)
            pltpu.async_copy(
                table_hbm.at[ridx_all.at[pl.ds(ci * CHUNK, CHUNK)]], rb, semr)
            pltpu.async_copy(
                table_hbm.at[cidx_all.at[pl.ds(ci * CHUNK, CHUNK)]], cb, semc)

        def wait(ci, b):
            vb, _, rb, cb, semv, semr, semc = bufs[b]
            pltpu.make_async_copy(vals_hbm.at[chunk_base + ci], vb,
                                  semv).wait()
            pltpu.make_async_copy(
                table_hbm.at[ridx_all.at[pl.ds(ci * CHUNK, CHUNK)]], rb,
                semr).wait()
            pltpu.make_async_copy(
                table_hbm.at[cidx_all.at[pl.ds(ci * CHUNK, CHUNK)]], cb,
                semc).wait()

        issue(0, 0)
        issue(1, 1)

        def outer(it, accs):
            for b in range(2):
                ci = 2 * it + b
                vb, ve, rb, cb = (bufs[b][0], bufs[b][1], bufs[b][2],
                                  bufs[b][3])
                wait(ci, b)

                for g in range(CHUNK // LANES):
                    vvg = vb[pl.ds(g * LANES, LANES)]
                    for l in range(LANES):
                        ve[pl.ds((g * LANES + l) * LANES, LANES)] = (
                            lax.gather(
                                vvg, jnp.full((LANES, 1), l, jnp.int32),
                                _SPLAT_DNUMS, slice_sizes=(1,),
                                mode=lax.GatherScatterMode.PROMISE_IN_BOUNDS))

                @pl.when(ci + 2 < NCHUNKS)
                def _():
                    issue(ci + 2, b)

                def nnz_body(j, accs):
                    vsp = ve[pl.ds(j * LANES, LANES)]
                    new = []
                    for s in range(n_slices):
                        r_s = rb[j, pl.ds(s * LANES, LANES)]
                        c_s = cb[j, pl.ds(s * LANES, LANES)]
                        new.append(accs[s] + vsp * r_s * c_s)
                    return tuple(new)

                accs = lax.fori_loop(0, CHUNK, nnz_body, accs, unroll=4)
            return accs

        zero = jnp.zeros((LANES,), jnp.float32)
        accs = lax.fori_loop(0, NCHUNKS // 2, outer,
                             tuple(zero for _ in range(n_slices)))
        total = accs[0]
        for s in range(1, n_slices):
            total = total + accs[s]
        out_v[...] = total
        pltpu.sync_copy(out_v, out_hbm.at[wid])

    return sc_loss


def kernel(vvecone, vvectwo, mmat_row, mmat_col, mmat_val):
    nbatch = vvecone.shape[0]
    info = plsc.get_sparse_core_info()
    nc, ns = info.num_cores, info.num_subcores
    nw = nc * ns

    table = _diff_t(vvecone, vvectwo)

    nnz = mmat_row.shape[0]
    tot_chunks = nw * NCHUNKS
    nnz_pad = tot_chunks * CHUNK
    pad = nnz_pad - nnz
    pad_idx = jnp.arange(pad, dtype=jnp.int32) % N

    rows = jnp.concatenate([mmat_row.astype(jnp.int32), pad_idx])
    cols = jnp.concatenate([mmat_col.astype(jnp.int32), pad_idx])
    vals = jnp.pad(mmat_val, (0, pad)).reshape(tot_chunks, CHUNK)

    parts = _make_sc_loss(nw, nc)(table, rows, cols, vals)
    return parts.sum() * (1.0 / (2.0 * nbatch))


# speedup vs baseline: 1.0336x; 1.0336x over previous
import functools

import jax
import jax.numpy as jnp
from jax import lax
from jax.experimental import pallas as pl
from jax.experimental.pallas import tpu as pltpu
from jax.experimental.pallas import tpu_sc as plsc

N = 16384
B = 256
LANES = 16
CHUNK = 64
NCHUNKS = 132

_SPLAT_DNUMS = lax.GatherDimensionNumbers(
    offset_dims=(), collapsed_slice_dims=(0,), start_index_map=(0,))


def _diff_t_body(a_ref, b_ref, o_ref):
    o_ref[...] = (a_ref[...] - b_ref[...]).T


def _diff_t(v1, v2):
    blk = 4096
    return pl.pallas_call(
        _diff_t_body,
        grid=(N // blk,),
        in_specs=[
            pl.BlockSpec((B, blk), lambda i: (0, i)),
            pl.BlockSpec((B, blk), lambda i: (0, i)),
        ],
        out_specs=pl.BlockSpec((blk, B), lambda i: (i, 0)),
        out_shape=jax.ShapeDtypeStruct((N, B), jnp.float32),
    )(v1, v2)


def _make_sc_loss(nw, nc):
    mesh = plsc.VectorSubcoreMesh(core_axis_name="c", subcore_axis_name="s")
    n_slices = B // LANES
    per_w = NCHUNKS * CHUNK

    @functools.partial(
        pl.kernel,
        mesh=mesh,
        out_type=jax.ShapeDtypeStruct((nw, LANES), jnp.float32),
        scratch_types=[
            pltpu.VMEM((per_w,), jnp.int32),
            pltpu.VMEM((per_w,), jnp.int32),
            pltpu.VMEM((CHUNK,), jnp.float32),
            pltpu.VMEM((CHUNK,), jnp.float32),
            pltpu.VMEM((CHUNK * LANES,), jnp.float32),
            pltpu.VMEM((CHUNK * LANES,), jnp.float32),
            pltpu.VMEM((CHUNK, B), jnp.float32),
            pltpu.VMEM((CHUNK, B), jnp.float32),
            pltpu.VMEM((CHUNK, B), jnp.float32),
            pltpu.VMEM((CHUNK, B), jnp.float32),
            pltpu.VMEM((LANES,), jnp.float32),
            pltpu.SemaphoreType.DMA,
            pltpu.SemaphoreType.DMA,
            pltpu.SemaphoreType.DMA,
            pltpu.SemaphoreType.DMA,
            pltpu.SemaphoreType.DMA,
            pltpu.SemaphoreType.DMA,
        ],
    )
    def sc_loss(table_hbm, rows_hbm, cols_hbm, vals_hbm, out_hbm,
                ridx_all, cidx_all, vraw0, vraw1, vexp0, vexp1,
                r0, c0, r1, c1, out_v,
                semv0, semr0, semc0, semv1, semr1, semc1):
        cid = lax.axis_index("c")
        sid = lax.axis_index("s")
        wid = sid * nc + cid
        elem_base = wid * per_w
        chunk_base = wid * NCHUNKS

        pltpu.sync_copy(rows_hbm.at[pl.ds(elem_base, per_w)], ridx_all)
        pltpu.sync_copy(cols_hbm.at[pl.ds(elem_base, per_w)], cidx_all)

        bufs = [(vraw0, vexp0, r0, c0, semv0, semr0, semc0),
                (vraw1, vexp1, r1, c1, semv1, semr1, semc1)]

        def issue(ci, b):
            vb, _, rb, cb, semv, semr, semc = bufs[b]
            pltpu.async_copy(vals_hbm.at[chunk_base + ci], vb, semv)
            pltpu.async_copy(
                table_hbm.at[ridx_all.at[pl.ds(ci * CHUNK, CHUNK)]], rb, semr)
            pltpu.async_copy(
                table_hbm.at[cidx_all.at[pl.ds(ci * CHUNK, CHUNK)]], cb, semc)

        def wait(ci, b):
            vb, _, rb, cb, semv, semr, semc = bufs[b]
            pltpu.make_async_copy(vals_hbm.at[chunk_base + ci], vb,
                                  semv).wait()
            pltpu.make_async_copy(
                table_hbm.at[ridx_all.at[pl.ds(ci * CHUNK, CHUNK)]], rb,
                semr).wait()
            pltpu.make_async_copy(
                table_hbm.at[cidx_all.at[pl.ds(ci * CHUNK, CHUNK)]], cb,
                semc).wait()

        issue(0, 0)
        issue(1, 1)

        def outer(it, accs):
            for b in range(2):
                ci = 2 * it + b
                vb, ve, rb, cb = (bufs[b][0], bufs[b][1], bufs[b][2],
                                  bufs[b][3])
                wait(ci, b)

                for g in range(CHUNK // LANES):
                    vvg = vb[pl.ds(g * LANES, LANES)]
                    for l in range(LANES):
                        ve[pl.ds((g * LANES + l) * LANES, LANES)] = (
                            lax.gather(
                                vvg, jnp.full((LANES, 1), l, jnp.int32),
                                _SPLAT_DNUMS, slice_sizes=(1,),
                                mode=lax.GatherScatterMode.PROMISE_IN_BOUNDS))

                def nnz_body(j, accs):
                    vsp = ve[pl.ds(j * LANES, LANES)]
                    new = []
                    for s in range(n_slices):
                        r_s = rb[j, pl.ds(s * LANES, LANES)]
                        c_s = cb[j, pl.ds(s * LANES, LANES)]
                        new.append(accs[s] + vsp * r_s * c_s)
                    return tuple(new)

                accs = lax.fori_loop(0, CHUNK, nnz_body, accs, unroll=4)

                @pl.when(ci + 2 < NCHUNKS)
                def _():
                    issue(ci + 2, b)
            return accs

        zero = jnp.zeros((LANES,), jnp.float32)
        accs = lax.fori_loop(0, NCHUNKS // 2, outer,
                             tuple(zero for _ in range(n_slices)))
        total = accs[0]
        for s in range(1, n_slices):
            total = total + accs[s]
        out_v[...] = total
        pltpu.sync_copy(out_v, out_hbm.at[wid])

    return sc_loss


def kernel(vvecone, vvectwo, mmat_row, mmat_col, mmat_val):
    nbatch = vvecone.shape[0]
    info = plsc.get_sparse_core_info()
    nc, ns = info.num_cores, info.num_subcores
    nw = nc * ns

    table = _diff_t(vvecone, vvectwo)

    nnz = mmat_row.shape[0]
    tot_chunks = nw * NCHUNKS
    nnz_pad = tot_chunks * CHUNK
    pad = nnz_pad - nnz
    pad_idx = jnp.arange(pad, dtype=jnp.int32) % N

    rows = jnp.concatenate([mmat_row.astype(jnp.int32), pad_idx])
    cols = jnp.concatenate([mmat_col.astype(jnp.int32), pad_idx])
    vals = jnp.pad(mmat_val, (0, pad)).reshape(tot_chunks, CHUNK)

    parts = _make_sc_loss(nw, nc)(table, rows, cols, vals)
    return parts.sum() * (1.0 / (2.0 * nbatch))


# speedup vs baseline: 1.0673x; 1.0326x over previous
import functools

import jax
import jax.numpy as jnp
from jax import lax
from jax.experimental import pallas as pl
from jax.experimental.pallas import tpu as pltpu
from jax.experimental.pallas import tpu_sc as plsc

N = 16384
B = 256
LANES = 16
CHUNK = 64
NCHUNKS = 132

_SPLAT_DNUMS = lax.GatherDimensionNumbers(
    offset_dims=(), collapsed_slice_dims=(0,), start_index_map=(0,))


def _diff_t_body(a_ref, b_ref, o_ref):
    o_ref[...] = (a_ref[...] - b_ref[...]).T


def _diff_t(v1, v2):
    blk = 4096
    return pl.pallas_call(
        _diff_t_body,
        grid=(N // blk,),
        in_specs=[
            pl.BlockSpec((B, blk), lambda i: (0, i)),
            pl.BlockSpec((B, blk), lambda i: (0, i)),
        ],
        out_specs=pl.BlockSpec((blk, B), lambda i: (i, 0)),
        out_shape=jax.ShapeDtypeStruct((N, B), jnp.float32),
    )(v1, v2)


def _make_sc_loss(nw, nc):
    mesh = plsc.VectorSubcoreMesh(core_axis_name="c", subcore_axis_name="s")
    n_slices = B // LANES
    per_w = NCHUNKS * CHUNK

    @functools.partial(
        pl.kernel,
        mesh=mesh,
        out_type=jax.ShapeDtypeStruct((nw, LANES), jnp.float32),
        scratch_types=[
            pltpu.VMEM((per_w,), jnp.int32),
            pltpu.VMEM((per_w,), jnp.int32),
            pltpu.VMEM((CHUNK,), jnp.float32),
            pltpu.VMEM((CHUNK,), jnp.float32),
            pltpu.VMEM((CHUNK,), jnp.float32),
            pltpu.VMEM((CHUNK * LANES,), jnp.float32),
            pltpu.VMEM((CHUNK * LANES,), jnp.float32),
            pltpu.VMEM((CHUNK * LANES,), jnp.float32),
            pltpu.VMEM((CHUNK, B), jnp.float32),
            pltpu.VMEM((CHUNK, B), jnp.float32),
            pltpu.VMEM((CHUNK, B), jnp.float32),
            pltpu.VMEM((CHUNK, B), jnp.float32),
            pltpu.VMEM((CHUNK, B), jnp.float32),
            pltpu.VMEM((CHUNK, B), jnp.float32),
            pltpu.VMEM((LANES,), jnp.float32),
            pltpu.SemaphoreType.DMA,
            pltpu.SemaphoreType.DMA,
            pltpu.SemaphoreType.DMA,
            pltpu.SemaphoreType.DMA,
            pltpu.SemaphoreType.DMA,
            pltpu.SemaphoreType.DMA,
            pltpu.SemaphoreType.DMA,
            pltpu.SemaphoreType.DMA,
            pltpu.SemaphoreType.DMA,
        ],
    )
    def sc_loss(table_hbm, rows_hbm, cols_hbm, vals_hbm, out_hbm,
                ridx_all, cidx_all, vraw0, vraw1, vraw2,
                vexp0, vexp1, vexp2,
                r0, c0, r1, c1, r2, c2, out_v,
                semv0, semr0, semc0, semv1, semr1, semc1,
                semv2, semr2, semc2):
        cid = lax.axis_index("c")
        sid = lax.axis_index("s")
        wid = sid * nc + cid
        elem_base = wid * per_w
        chunk_base = wid * NCHUNKS

        pltpu.sync_copy(rows_hbm.at[pl.ds(elem_base, per_w)], ridx_all)
        pltpu.sync_copy(cols_hbm.at[pl.ds(elem_base, per_w)], cidx_all)

        bufs = [(vraw0, vexp0, r0, c0, semv0, semr0, semc0),
                (vraw1, vexp1, r1, c1, semv1, semr1, semc1),
                (vraw2, vexp2, r2, c2, semv2, semr2, semc2)]

        def issue(ci, b):
            vb, _, rb, cb, semv, semr, semc = bufs[b]
            pltpu.async_copy(vals_hbm.at[chunk_base + ci], vb, semv)
            pltpu.async_copy(
                table_hbm.at[ridx_all.at[pl.ds(ci * CHUNK, CHUNK)]], rb, semr)
            pltpu.async_copy(
                table_hbm.at[cidx_all.at[pl.ds(ci * CHUNK, CHUNK)]], cb, semc)

        def wait(ci, b):
            vb, _, rb, cb, semv, semr, semc = bufs[b]
            pltpu.make_async_copy(vals_hbm.at[chunk_base + ci], vb,
                                  semv).wait()
            pltpu.make_async_copy(
                table_hbm.at[ridx_all.at[pl.ds(ci * CHUNK, CHUNK)]], rb,
                semr).wait()
            pltpu.make_async_copy(
                table_hbm.at[cidx_all.at[pl.ds(ci * CHUNK, CHUNK)]], cb,
                semc).wait()

        issue(0, 0)
        issue(1, 1)
        issue(2, 2)

        def outer(it, accs):
            for b in range(3):
                ci = 3 * it + b
                vb, ve, rb, cb = (bufs[b][0], bufs[b][1], bufs[b][2],
                                  bufs[b][3])
                wait(ci, b)

                for g in range(CHUNK // LANES):
                    vvg = vb[pl.ds(g * LANES, LANES)]
                    for l in range(LANES):
                        ve[pl.ds((g * LANES + l) * LANES, LANES)] = (
                            lax.gather(
                                vvg, jnp.full((LANES, 1), l, jnp.int32),
                                _SPLAT_DNUMS, slice_sizes=(1,),
                                mode=lax.GatherScatterMode.PROMISE_IN_BOUNDS))

                def nnz_body(j, accs):
                    vsp = ve[pl.ds(j * LANES, LANES)]
                    new = []
                    for s in range(n_slices):
                        r_s = rb[j, pl.ds(s * LANES, LANES)]
                        c_s = cb[j, pl.ds(s * LANES, LANES)]
                        new.append(accs[s] + vsp * r_s * c_s)
                    return tuple(new)

                accs = lax.fori_loop(0, CHUNK, nnz_body, accs, unroll=4)

                @pl.when(ci + 3 < NCHUNKS)
                def _():
                    issue(ci + 3, b)
            return accs

        zero = jnp.zeros((LANES,), jnp.float32)
        accs = lax.fori_loop(0, NCHUNKS // 3, outer,
                             tuple(zero for _ in range(n_slices)))
        total = accs[0]
        for s in range(1, n_slices):
            total = total + accs[s]
        out_v[...] = total
        pltpu.sync_copy(out_v, out_hbm.at[wid])

    return sc_loss


def kernel(vvecone, vvectwo, mmat_row, mmat_col, mmat_val):
    nbatch = vvecone.shape[0]
    info = plsc.get_sparse_core_info()
    nc, ns = info.num_cores, info.num_subcores
    nw = nc * ns

    table = _diff_t(vvecone, vvectwo)

    nnz = mmat_row.shape[0]
    tot_chunks = nw * NCHUNKS
    nnz_pad = tot_chunks * CHUNK
    pad = nnz_pad - nnz
    pad_idx = jnp.arange(pad, dtype=jnp.int32) % N

    rows = jnp.concatenate([mmat_row.astype(jnp.int32), pad_idx])
    cols = jnp.concatenate([mmat_col.astype(jnp.int32), pad_idx])
    vals = jnp.pad(mmat_val, (0, pad)).reshape(tot_chunks, CHUNK)

    parts = _make_sc_loss(nw, nc)(table, rows, cols, vals)
    return parts.sum() * (1.0 / (2.0 * nbatch))
